# baseline (speedup 1.0000x reference)
import functools

import jax
import jax.numpy as jnp
from jax import lax
from jax.experimental import pallas as pl
from jax.experimental.pallas import tpu as pltpu
from jax.experimental.pallas import tpu_sc as plsc

N = 10000
E = 320000
EL = 200000
D = 128
H = 128

NP_ = 10240
EP = 327680
ELP = 204800
CH = 128
NCH_E = EP // (32 * CH)
NCH_L = ELP // (32 * CH)
TROWS = NP_ // 16

_mesh = plsc.VectorSubcoreMesh(core_axis_name="c", subcore_axis_name="s")


def _zero_fill(ref, nrows):
    zz = jnp.zeros((16,), jnp.float32)

    def body(i, _):
        for j in range(8):
            ref[i, pl.ds(j * 16, 16)] = zz
        return 0

    lax.fori_loop(0, nrows, body, 0)


@functools.partial(
    pl.kernel,
    out_type=jax.ShapeDtypeStruct((2, 16, TROWS), jnp.float32),
    mesh=_mesh,
    scratch_types=[
        pltpu.VMEM((CH,), jnp.int32),
        pltpu.VMEM((CH,), jnp.float32),
        pltpu.MemorySpace.VMEM_SHARED((NP_,), jnp.float32),
        pltpu.SemaphoreType.DMA,
    ],
)
def _sc_degree(dst_hbm, out_hbm, idx_v, ones_v, deg_sh, sem):
    c = lax.axis_index("c")
    s = lax.axis_index("s")
    w = c * 16 + s

    one = jnp.ones((16,), jnp.float32)
    zz = jnp.zeros((16,), jnp.float32)
    for j in range(CH // 16):
        ones_v[pl.ds(j * 16, 16)] = zz
    for k in range(TROWS // CH):
        pltpu.sync_copy(ones_v, deg_sh.at[pl.ds(s * TROWS + k * CH, CH)])
    for j in range(CH // 16):
        ones_v[pl.ds(j * 16, 16)] = one

    plsc.subcore_barrier()

    def body(i, _):
        base = (w * NCH_E + i) * CH
        pltpu.sync_copy(dst_hbm.at[pl.ds(base, CH)], idx_v)
        pltpu.sync_copy(ones_v, deg_sh.at[idx_v], add=True)
        return 0

    lax.fori_loop(0, NCH_E, body, 0)

    plsc.subcore_barrier()
    pltpu.sync_copy(deg_sh.at[pl.ds(s * TROWS, TROWS)], out_hbm.at[c, s])


@functools.partial(
    pl.kernel,
    out_type=jax.ShapeDtypeStruct((2, 16, TROWS, D), jnp.float32),
    mesh=_mesh,
    scratch_types=[
        pltpu.VMEM((2, CH), jnp.int32),
        pltpu.VMEM((2, CH), jnp.int32),
        pltpu.VMEM((2, CH, D), jnp.float32),
        pltpu.MemorySpace.VMEM_SHARED((NP_, D), jnp.float32),
        pltpu.SemaphoreType.DMA,
    ],
)
def _sc_spmm(src_hbm, dst_hbm, hp_hbm, out_hbm, sidx_v, didx_v, rows_v,
             agg_sh, sem):
    c = lax.axis_index("c")
    s = lax.axis_index("s")
    w = c * 16 + s

    @pl.when(c == 0)
    def _():
        pltpu.sync_copy(hp_hbm.at[pl.ds(s * TROWS, TROWS)],
                        agg_sh.at[pl.ds(s * TROWS, TROWS)])

    @pl.when(c == 1)
    def _():
        _zero_fill(rows_v.at[0], CH)
        for k in range(TROWS // CH):
            pltpu.sync_copy(rows_v.at[0],
                            agg_sh.at[pl.ds(s * TROWS + k * CH, CH)])

    plsc.subcore_barrier()

    def body(g, _):
        cps = []
        for b in range(2):
            base = (w * NCH_E + g * 2 + b) * CH
            pltpu.sync_copy(src_hbm.at[pl.ds(base, CH)], sidx_v.at[b])
            cps.append(
                pltpu.async_copy(hp_hbm.at[sidx_v.at[b]], rows_v.at[b], sem))
            pltpu.sync_copy(dst_hbm.at[pl.ds(base, CH)], didx_v.at[b])
        for b in range(2):
            cps[b].wait()
        for b in range(2):
            pltpu.sync_copy(rows_v.at[b], agg_sh.at[didx_v.at[b]], add=True)
        return 0

    lax.fori_loop(0, NCH_E // 2, body, 0)

    plsc.subcore_barrier()
    pltpu.sync_copy(agg_sh.at[pl.ds(s * TROWS, TROWS)], out_hbm.at[c, s])


@functools.partial(
    pl.kernel,
    out_type=jax.ShapeDtypeStruct((ELP, D), jnp.float32),
    mesh=_mesh,
    scratch_types=[
        pltpu.VMEM((CH,), jnp.int32),
        pltpu.VMEM((CH,), jnp.int32),
        pltpu.VMEM((CH,), jnp.int32),
        pltpu.VMEM((CH, D), jnp.float32),
        pltpu.VMEM((CH, D), jnp.float32),
        pltpu.MemorySpace.VMEM_SHARED((16 * CH, D), jnp.float32),
        pltpu.SemaphoreType.DMA,
        pltpu.SemaphoreType.DMA,
    ],
)
def _sc_decode(s_hbm, d_hbm, a_hbm, b_hbm, sum_hbm,
               sidx_v, didx_v, iota_v, rowsa_v, rowsb_v, sum_sh, sema, semb):
    c = lax.axis_index("c")
    s = lax.axis_index("s")
    w = c * 16 + s

    lanes = lax.iota(jnp.int32, 16)
    for j in range(CH // 16):
        iota_v[pl.ds(j * 16, 16)] = lanes + (s * CH + j * 16)

    def body(i, _):
        base = (w * NCH_L + i) * CH
        pltpu.sync_copy(s_hbm.at[pl.ds(base, CH)], sidx_v)
        pltpu.sync_copy(d_hbm.at[pl.ds(base, CH)], didx_v)
        ca = pltpu.async_copy(a_hbm.at[sidx_v], rowsa_v, sema)
        cb = pltpu.async_copy(b_hbm.at[didx_v], rowsb_v, semb)
        ca.wait()
        pltpu.sync_copy(rowsa_v, sum_sh.at[pl.ds(s * CH, CH)])
        cb.wait()
        pltpu.sync_copy(rowsb_v, sum_sh.at[iota_v], add=True)
        pltpu.sync_copy(sum_sh.at[pl.ds(s * CH, CH)],
                        sum_hbm.at[pl.ds(base, CH)])
        return 0

    lax.fori_loop(0, NCH_L, body, 0)


_BN = 1024


def _tc_call(body, grid, in_specs, out_specs, out_shape):
    return pl.pallas_call(body, grid=grid, in_specs=in_specs,
                          out_specs=out_specs, out_shape=out_shape)


def _dinv_of(deg_ref):
    deg = deg_ref[0, :] + deg_ref[1, :] + 1.0
    return lax.rsqrt(deg)


def _tc_h1p(xp, w1, degp):
    def body(x_ref, w_ref, deg_ref, o_ref):
        dinv = _dinv_of(deg_ref)
        h = jnp.dot(x_ref[...], w_ref[...], preferred_element_type=jnp.float32)
        o_ref[...] = h * dinv[:, None]

    return _tc_call(
        body, (NP_ // _BN,),
        [pl.BlockSpec((_BN, D), lambda i: (i, 0)),
         pl.BlockSpec((D, H), lambda i: (0, 0)),
         pl.BlockSpec((2, _BN), lambda i: (0, i))],
        pl.BlockSpec((_BN, H), lambda i: (i, 0)),
        jax.ShapeDtypeStruct((NP_, H), jnp.float32),
    )(xp, w1, degp)


def _tc_mid(parts, degp, b1r, w2):
    def body(p_ref, deg_ref, b_ref, w_ref, o_ref):
        dinv = _dinv_of(deg_ref)
        z = (p_ref[0] + p_ref[1]) * dinv[:, None] + b_ref[...]
        z = jnp.maximum(z, 0.0)
        h = jnp.dot(z, w_ref[...], preferred_element_type=jnp.float32)
        o_ref[...] = h * dinv[:, None]

    return _tc_call(
        body, (NP_ // _BN,),
        [pl.BlockSpec((2, _BN, H), lambda i: (0, i, 0)),
         pl.BlockSpec((2, _BN), lambda i: (0, i)),
         pl.BlockSpec((1, H), lambda i: (0, 0)),
         pl.BlockSpec((H, H), lambda i: (0, 0))],
        pl.BlockSpec((_BN, H), lambda i: (i, 0)),
        jax.ShapeDtypeStruct((NP_, H), jnp.float32),
    )(parts, degp, b1r, w2)


def _tc_ab(parts, degp, b2r, bc1r, wc1t, wc1b):
    def body(p_ref, deg_ref, b_ref, bc_ref, wt_ref, wb_ref, a_ref, bo_ref):
        dinv = _dinv_of(deg_ref)
        z = (p_ref[0] + p_ref[1]) * dinv[:, None] + b_ref[...]
        a_ref[...] = (jnp.dot(z, wt_ref[...], preferred_element_type=jnp.float32)
                      + bc_ref[...])
        bo_ref[...] = jnp.dot(z, wb_ref[...], preferred_element_type=jnp.float32)

    return _tc_call(
        body, (NP_ // _BN,),
        [pl.BlockSpec((2, _BN, H), lambda i: (0, i, 0)),
         pl.BlockSpec((2, _BN), lambda i: (0, i)),
         pl.BlockSpec((1, H), lambda i: (0, 0)),
         pl.BlockSpec((1, H), lambda i: (0, 0)),
         pl.BlockSpec((H, H), lambda i: (0, 0)),
         pl.BlockSpec((H, H), lambda i: (0, 0))],
        [pl.BlockSpec((_BN, H), lambda i: (i, 0)),
         pl.BlockSpec((_BN, H), lambda i: (i, 0))],
        [jax.ShapeDtypeStruct((NP_, H), jnp.float32),
         jax.ShapeDtypeStruct((NP_, H), jnp.float32)],
    )(parts, degp, b2r, bc1r, wc1t, wc1b)


def _tc_out(hsum, wc2t):
    def body(h_ref, w_ref, o_ref):
        h = jnp.maximum(h_ref[...], 0.0)
        o_ref[...] = lax.dot_general(
            w_ref[...], h, (((1,), (1,)), ((), ())),
            preferred_element_type=jnp.float32)

    return _tc_call(
        body, (ELP // _BN,),
        [pl.BlockSpec((_BN, H), lambda i: (i, 0)),
         pl.BlockSpec((8, H), lambda i: (0, 0))],
        pl.BlockSpec((8, _BN), lambda i: (0, i)),
        jax.ShapeDtypeStruct((8, ELP), jnp.float32),
    )(hsum, wc2t)


def kernel(x, edge_index, edge_label_index, W1, b1, W2, b2, Wc1, bc1, Wc2,
           bc2):
    f32 = jnp.float32
    xp = jnp.pad(x, ((0, NP_ - N), (0, 0)))
    pad_e = (jnp.arange(EP - E, dtype=jnp.int32) % (NP_ - N)) + N
    srcp = jnp.concatenate([edge_index[0], pad_e])
    dstp = jnp.concatenate([edge_index[1], pad_e])
    pad_l = (jnp.arange(ELP - EL, dtype=jnp.int32) % (NP_ - N)) + N
    sp = jnp.concatenate([edge_label_index[0], pad_l])
    dp = jnp.concatenate([edge_label_index[1], pad_l])

    b1r = b1.reshape(1, H)
    b2r = b2.reshape(1, H)
    bc1r = bc1.reshape(1, H)
    wc1t = Wc1[:H]
    wc1b = Wc1[H:]
    wc2t = jnp.pad(Wc2.T, ((0, 6), (0, 0)))

    degp = _sc_degree(dstp).reshape(2, NP_)

    h1p = _tc_h1p(xp, W1, degp)
    parts1 = _sc_spmm(srcp, dstp, h1p).reshape(2, NP_, D)
    h2p = _tc_mid(parts1, degp, b1r, W2)
    parts2 = _sc_spmm(srcp, dstp, h2p).reshape(2, NP_, D)
    a_nodes, b_nodes = _tc_ab(parts2, degp, b2r, bc1r, wc1t, wc1b)
    hsum = _sc_decode(sp, dp, a_nodes, b_nodes)
    outp = _tc_out(hsum, wc2t)
    return (outp[:2, :EL].T + bc2[None, :]).astype(f32)


# speedup vs baseline: 1.2102x; 1.2102x over previous
import functools

import jax
import jax.numpy as jnp
from jax import lax
from jax.experimental import pallas as pl
from jax.experimental.pallas import tpu as pltpu
from jax.experimental.pallas import tpu_sc as plsc

N = 10000
E = 320000
EL = 200000
D = 128
H = 128

NP_ = 10240
EP = 327680
ELP = 204800
CH = 128
NCH_E = EP // (32 * CH)
NCH_L = ELP // (32 * CH)
TROWS = NP_ // 16

_mesh = plsc.VectorSubcoreMesh(core_axis_name="c", subcore_axis_name="s")


def _zero_fill(ref, nrows):
    zz = jnp.zeros((16,), jnp.float32)

    def body(i, _):
        for j in range(8):
            ref[i, pl.ds(j * 16, 16)] = zz
        return 0

    lax.fori_loop(0, nrows, body, 0)


@functools.partial(
    pl.kernel,
    out_type=jax.ShapeDtypeStruct((2, 16, TROWS), jnp.float32),
    mesh=_mesh,
    scratch_types=[
        pltpu.VMEM((CH,), jnp.int32),
        pltpu.VMEM((CH,), jnp.float32),
        pltpu.MemorySpace.VMEM_SHARED((NP_,), jnp.float32),
        pltpu.SemaphoreType.DMA,
    ],
)
def _sc_degree(dst_hbm, out_hbm, idx_v, ones_v, deg_sh, sem):
    c = lax.axis_index("c")
    s = lax.axis_index("s")
    w = c * 16 + s

    one = jnp.ones((16,), jnp.float32)
    zz = jnp.zeros((16,), jnp.float32)
    for j in range(CH // 16):
        ones_v[pl.ds(j * 16, 16)] = zz
    for k in range(TROWS // CH):
        pltpu.sync_copy(ones_v, deg_sh.at[pl.ds(s * TROWS + k * CH, CH)])
    for j in range(CH // 16):
        ones_v[pl.ds(j * 16, 16)] = one

    plsc.subcore_barrier()

    def body(i, _):
        base = (w * NCH_E + i) * CH
        pltpu.sync_copy(dst_hbm.at[pl.ds(base, CH)], idx_v)
        pltpu.sync_copy(ones_v, deg_sh.at[idx_v], add=True)
        return 0

    lax.fori_loop(0, NCH_E, body, 0)

    plsc.subcore_barrier()
    pltpu.sync_copy(deg_sh.at[pl.ds(s * TROWS, TROWS)], out_hbm.at[c, s])


@functools.partial(
    pl.kernel,
    out_type=jax.ShapeDtypeStruct((2, 16, TROWS, D), jnp.float32),
    mesh=_mesh,
    scratch_types=[
        pltpu.VMEM((2, CH), jnp.int32),
        pltpu.VMEM((2, CH), jnp.int32),
        pltpu.VMEM((2, CH, D), jnp.float32),
        pltpu.MemorySpace.VMEM_SHARED((NP_, D), jnp.float32),
        pltpu.SemaphoreType.DMA,
    ],
)
def _sc_spmm(src_hbm, dst_hbm, hp_hbm, out_hbm, sidx_v, didx_v, rows_v,
             agg_sh, sem):
    c = lax.axis_index("c")
    s = lax.axis_index("s")
    w = c * 16 + s

    @pl.when(c == 0)
    def _():
        pltpu.sync_copy(hp_hbm.at[pl.ds(s * TROWS, TROWS)],
                        agg_sh.at[pl.ds(s * TROWS, TROWS)])

    @pl.when(c == 1)
    def _():
        _zero_fill(rows_v.at[0], CH)
        for k in range(TROWS // CH):
            pltpu.sync_copy(rows_v.at[0],
                            agg_sh.at[pl.ds(s * TROWS + k * CH, CH)])

    plsc.subcore_barrier()

    for b in range(2):
        base = (w * NCH_E + b) * CH
        pltpu.sync_copy(src_hbm.at[pl.ds(base, CH)], sidx_v.at[b])
        pltpu.async_copy(hp_hbm.at[sidx_v.at[b]], rows_v.at[b], sem)
        pltpu.sync_copy(dst_hbm.at[pl.ds(base, CH)], didx_v.at[b])

    def body(g, _):
        for b in range(2):
            pltpu.make_async_copy(hp_hbm.at[pl.ds(0, CH)], rows_v.at[b],
                                  sem).wait()
            pltpu.sync_copy(rows_v.at[b], agg_sh.at[didx_v.at[b]], add=True)
            nxt = 2 * g + b + 2

            @pl.when(nxt < NCH_E)
            def _():
                base = (w * NCH_E + nxt) * CH
                pltpu.sync_copy(src_hbm.at[pl.ds(base, CH)], sidx_v.at[b])
                pltpu.async_copy(hp_hbm.at[sidx_v.at[b]], rows_v.at[b], sem)
                pltpu.sync_copy(dst_hbm.at[pl.ds(base, CH)], didx_v.at[b])
        return 0

    lax.fori_loop(0, NCH_E // 2, body, 0)

    plsc.subcore_barrier()
    pltpu.sync_copy(agg_sh.at[pl.ds(s * TROWS, TROWS)], out_hbm.at[c, s])


@functools.partial(
    pl.kernel,
    out_type=jax.ShapeDtypeStruct((ELP, D), jnp.float32),
    mesh=_mesh,
    scratch_types=[
        pltpu.VMEM((2, CH), jnp.int32),
        pltpu.VMEM((2, CH), jnp.int32),
        pltpu.VMEM((2, CH, D), jnp.float32),
        pltpu.SemaphoreType.DMA,
    ],
)
def _sc_decode(s_hbm, d_hbm, a_hbm, b_hbm, o_hbm,
               sidx_v, didx_v, rows_v, sema):
    c = lax.axis_index("c")
    s = lax.axis_index("s")
    w = c * 16 + s

    for b in range(2):
        base = (w * NCH_L + b) * CH
        pltpu.sync_copy(s_hbm.at[pl.ds(base, CH)], sidx_v.at[b])
        pltpu.async_copy(a_hbm.at[sidx_v.at[b]], rows_v.at[b], sema)
        pltpu.sync_copy(d_hbm.at[pl.ds(base, CH)], didx_v.at[b])

    def body(g, _):
        for b in range(2):
            base = (w * NCH_L + 2 * g + b) * CH
            pltpu.make_async_copy(a_hbm.at[pl.ds(0, CH)], rows_v.at[b],
                                  sema).wait()
            pltpu.sync_copy(b_hbm.at[didx_v.at[b]], rows_v.at[b], add=True)
            pltpu.sync_copy(rows_v.at[b], o_hbm.at[pl.ds(base, CH)])
            nxt = 2 * g + b + 2

            @pl.when(nxt < NCH_L)
            def _():
                nbase = (w * NCH_L + nxt) * CH
                pltpu.sync_copy(s_hbm.at[pl.ds(nbase, CH)], sidx_v.at[b])
                pltpu.async_copy(a_hbm.at[sidx_v.at[b]], rows_v.at[b], sema)
                pltpu.sync_copy(d_hbm.at[pl.ds(nbase, CH)], didx_v.at[b])
        return 0

    lax.fori_loop(0, NCH_L // 2, body, 0)


_BN = 1024


def _tc_call(body, grid, in_specs, out_specs, out_shape):
    return pl.pallas_call(body, grid=grid, in_specs=in_specs,
                          out_specs=out_specs, out_shape=out_shape)


def _dinv_of(deg_ref):
    deg = deg_ref[0, :] + deg_ref[1, :] + 1.0
    return lax.rsqrt(deg)


def _tc_h1p(xp, w1, degp):
    def body(x_ref, w_ref, deg_ref, o_ref):
        dinv = _dinv_of(deg_ref)
        h = jnp.dot(x_ref[...], w_ref[...], preferred_element_type=jnp.float32)
        o_ref[...] = h * dinv[:, None]

    return _tc_call(
        body, (NP_ // _BN,),
        [pl.BlockSpec((_BN, D), lambda i: (i, 0)),
         pl.BlockSpec((D, H), lambda i: (0, 0)),
         pl.BlockSpec((2, _BN), lambda i: (0, i))],
        pl.BlockSpec((_BN, H), lambda i: (i, 0)),
        jax.ShapeDtypeStruct((NP_, H), jnp.float32),
    )(xp, w1, degp)


def _tc_mid(parts, degp, b1r, w2):
    def body(p_ref, deg_ref, b_ref, w_ref, o_ref):
        dinv = _dinv_of(deg_ref)
        z = (p_ref[0] + p_ref[1]) * dinv[:, None] + b_ref[...]
        z = jnp.maximum(z, 0.0)
        h = jnp.dot(z, w_ref[...], preferred_element_type=jnp.float32)
        o_ref[...] = h * dinv[:, None]

    return _tc_call(
        body, (NP_ // _BN,),
        [pl.BlockSpec((2, _BN, H), lambda i: (0, i, 0)),
         pl.BlockSpec((2, _BN), lambda i: (0, i)),
         pl.BlockSpec((1, H), lambda i: (0, 0)),
         pl.BlockSpec((H, H), lambda i: (0, 0))],
        pl.BlockSpec((_BN, H), lambda i: (i, 0)),
        jax.ShapeDtypeStruct((NP_, H), jnp.float32),
    )(parts, degp, b1r, w2)


def _tc_ab(parts, degp, b2r, bc1r, wc1t, wc1b):
    def body(p_ref, deg_ref, b_ref, bc_ref, wt_ref, wb_ref, a_ref, bo_ref):
        dinv = _dinv_of(deg_ref)
        z = (p_ref[0] + p_ref[1]) * dinv[:, None] + b_ref[...]
        a_ref[...] = (jnp.dot(z, wt_ref[...], preferred_element_type=jnp.float32)
                      + bc_ref[...])
        bo_ref[...] = jnp.dot(z, wb_ref[...], preferred_element_type=jnp.float32)

    return _tc_call(
        body, (NP_ // _BN,),
        [pl.BlockSpec((2, _BN, H), lambda i: (0, i, 0)),
         pl.BlockSpec((2, _BN), lambda i: (0, i)),
         pl.BlockSpec((1, H), lambda i: (0, 0)),
         pl.BlockSpec((1, H), lambda i: (0, 0)),
         pl.BlockSpec((H, H), lambda i: (0, 0)),
         pl.BlockSpec((H, H), lambda i: (0, 0))],
        [pl.BlockSpec((_BN, H), lambda i: (i, 0)),
         pl.BlockSpec((_BN, H), lambda i: (i, 0))],
        [jax.ShapeDtypeStruct((NP_, H), jnp.float32),
         jax.ShapeDtypeStruct((NP_, H), jnp.float32)],
    )(parts, degp, b2r, bc1r, wc1t, wc1b)


def _tc_out(hsum, wc2t):
    def body(h_ref, w_ref, o_ref):
        h = jnp.maximum(h_ref[...], 0.0)
        o_ref[...] = lax.dot_general(
            w_ref[...], h, (((1,), (1,)), ((), ())),
            preferred_element_type=jnp.float32)

    return _tc_call(
        body, (ELP // _BN,),
        [pl.BlockSpec((_BN, H), lambda i: (i, 0)),
         pl.BlockSpec((8, H), lambda i: (0, 0))],
        pl.BlockSpec((8, _BN), lambda i: (0, i)),
        jax.ShapeDtypeStruct((8, ELP), jnp.float32),
    )(hsum, wc2t)


def kernel(x, edge_index, edge_label_index, W1, b1, W2, b2, Wc1, bc1, Wc2,
           bc2):
    f32 = jnp.float32
    xp = jnp.pad(x, ((0, NP_ - N), (0, 0)))
    pad_e = (jnp.arange(EP - E, dtype=jnp.int32) % (NP_ - N)) + N
    srcp = jnp.concatenate([edge_index[0], pad_e])
    dstp = jnp.concatenate([edge_index[1], pad_e])
    pad_l = (jnp.arange(ELP - EL, dtype=jnp.int32) % (NP_ - N)) + N
    sp = jnp.concatenate([edge_label_index[0], pad_l])
    dp = jnp.concatenate([edge_label_index[1], pad_l])

    b1r = b1.reshape(1, H)
    b2r = b2.reshape(1, H)
    bc1r = bc1.reshape(1, H)
    wc1t = Wc1[:H]
    wc1b = Wc1[H:]
    wc2t = jnp.pad(Wc2.T, ((0, 6), (0, 0)))

    degp = _sc_degree(dstp).reshape(2, NP_)

    h1p = _tc_h1p(xp, W1, degp)
    parts1 = _sc_spmm(srcp, dstp, h1p).reshape(2, NP_, D)
    h2p = _tc_mid(parts1, degp, b1r, W2)
    parts2 = _sc_spmm(srcp, dstp, h2p).reshape(2, NP_, D)
    a_nodes, b_nodes = _tc_ab(parts2, degp, b2r, bc1r, wc1t, wc1b)
    hsum = _sc_decode(sp, dp, a_nodes, b_nodes)
    outp = _tc_out(hsum, wc2t)
    return (outp[:2, :EL].T + bc2[None, :]).astype(f32)


# speedup vs baseline: 1.2694x; 1.0490x over previous
import functools

import jax
import jax.numpy as jnp
from jax import lax
from jax.experimental import pallas as pl
from jax.experimental.pallas import tpu as pltpu
from jax.experimental.pallas import tpu_sc as plsc

N = 10000
E = 320000
EL = 200000
D = 128
H = 128

NP_ = 10240
EP = 327680
ELP = 204800
CH = 128
NCH_E = EP // (32 * CH)
NCH_L = ELP // (32 * CH)
TROWS = NP_ // 16

_mesh = plsc.VectorSubcoreMesh(core_axis_name="c", subcore_axis_name="s")


def _zero_fill(ref, nrows):
    zz = jnp.zeros((16,), jnp.float32)

    def body(i, _):
        for j in range(8):
            ref[i, pl.ds(j * 16, 16)] = zz
        return 0

    lax.fori_loop(0, nrows, body, 0)


@functools.partial(
    pl.kernel,
    out_type=jax.ShapeDtypeStruct((2, 16, TROWS), jnp.float32),
    mesh=_mesh,
    scratch_types=[
        pltpu.VMEM((CH,), jnp.int32),
        pltpu.VMEM((CH,), jnp.float32),
        pltpu.MemorySpace.VMEM_SHARED((NP_,), jnp.float32),
        pltpu.SemaphoreType.DMA,
    ],
)
def _sc_degree(dst_hbm, out_hbm, idx_v, ones_v, deg_sh, sem):
    c = lax.axis_index("c")
    s = lax.axis_index("s")
    w = c * 16 + s

    one = jnp.ones((16,), jnp.float32)
    zz = jnp.zeros((16,), jnp.float32)
    for j in range(CH // 16):
        ones_v[pl.ds(j * 16, 16)] = zz
    for k in range(TROWS // CH):
        pltpu.sync_copy(ones_v, deg_sh.at[pl.ds(s * TROWS + k * CH, CH)])
    for j in range(CH // 16):
        ones_v[pl.ds(j * 16, 16)] = one

    plsc.subcore_barrier()

    def body(i, _):
        base = (w * NCH_E + i) * CH
        pltpu.sync_copy(dst_hbm.at[pl.ds(base, CH)], idx_v)
        pltpu.sync_copy(ones_v, deg_sh.at[idx_v], add=True)
        return 0

    lax.fori_loop(0, NCH_E, body, 0)

    plsc.subcore_barrier()
    pltpu.sync_copy(deg_sh.at[pl.ds(s * TROWS, TROWS)], out_hbm.at[c, s])


@functools.partial(
    pl.kernel,
    out_type=jax.ShapeDtypeStruct((2, 16, TROWS, D), jnp.float32),
    mesh=_mesh,
    scratch_types=[
        pltpu.VMEM((2, CH), jnp.int32),
        pltpu.VMEM((2, CH), jnp.int32),
        pltpu.VMEM((2, CH, D), jnp.float32),
        pltpu.MemorySpace.VMEM_SHARED((NP_, D), jnp.float32),
        pltpu.SemaphoreType.DMA,
    ],
)
def _sc_spmm(src_hbm, dst_hbm, hp_hbm, out_hbm, sidx_v, didx_v, rows_v,
             agg_sh, sem):
    c = lax.axis_index("c")
    s = lax.axis_index("s")
    w = c * 16 + s

    @pl.when(c == 0)
    def _():
        pltpu.sync_copy(hp_hbm.at[pl.ds(s * TROWS, TROWS)],
                        agg_sh.at[pl.ds(s * TROWS, TROWS)])

    @pl.when(c == 1)
    def _():
        _zero_fill(rows_v.at[0], CH)
        for k in range(TROWS // CH):
            pltpu.sync_copy(rows_v.at[0],
                            agg_sh.at[pl.ds(s * TROWS + k * CH, CH)])

    plsc.subcore_barrier()

    for b in range(2):
        base = (w * NCH_E + b) * CH
        pltpu.sync_copy(src_hbm.at[pl.ds(base, CH)], sidx_v.at[b])
        pltpu.async_copy(hp_hbm.at[sidx_v.at[b]], rows_v.at[b], sem)
        pltpu.sync_copy(dst_hbm.at[pl.ds(base, CH)], didx_v.at[b])

    def body(g, _):
        for b in range(2):
            pltpu.make_async_copy(hp_hbm.at[pl.ds(0, CH)], rows_v.at[b],
                                  sem).wait()
            pltpu.sync_copy(rows_v.at[b], agg_sh.at[didx_v.at[b]], add=True)
            nxt = 2 * g + b + 2

            @pl.when(nxt < NCH_E)
            def _():
                base = (w * NCH_E + nxt) * CH
                pltpu.sync_copy(src_hbm.at[pl.ds(base, CH)], sidx_v.at[b])
                pltpu.async_copy(hp_hbm.at[sidx_v.at[b]], rows_v.at[b], sem)
                pltpu.sync_copy(dst_hbm.at[pl.ds(base, CH)], didx_v.at[b])
        return 0

    lax.fori_loop(0, NCH_E // 2, body, 0)

    plsc.subcore_barrier()
    pltpu.sync_copy(agg_sh.at[pl.ds(s * TROWS, TROWS)], out_hbm.at[c, s])


@functools.partial(
    pl.kernel,
    out_type=jax.ShapeDtypeStruct((ELP, D), jnp.float32),
    mesh=_mesh,
    scratch_types=[
        pltpu.VMEM((2, CH), jnp.int32),
        pltpu.VMEM((2, CH), jnp.int32),
        pltpu.VMEM((2, CH, D), jnp.float32),
        pltpu.SemaphoreType.DMA,
        pltpu.SemaphoreType.DMA,
    ],
)
def _sc_decode(s_hbm, d_hbm, a_hbm, b_hbm, o_hbm,
               sidx_v, didx_v, rows_v, sema, semb):
    c = lax.axis_index("c")
    s = lax.axis_index("s")
    w = c * 16 + s

    for b in range(2):
        base = (w * NCH_L + b) * CH
        pltpu.sync_copy(s_hbm.at[pl.ds(base, CH)], sidx_v.at[b])
        pltpu.async_copy(a_hbm.at[sidx_v.at[b]], rows_v.at[b], sema)
        pltpu.sync_copy(d_hbm.at[pl.ds(base, CH)], didx_v.at[b])

    def body(g, _):
        for b in range(2):
            pltpu.make_async_copy(a_hbm.at[pl.ds(0, CH)], rows_v.at[b],
                                  sema).wait()
            pltpu.async_copy(b_hbm.at[didx_v.at[b]], rows_v.at[b], semb,
                             add=True)
        for b in range(2):
            base = (w * NCH_L + 2 * g + b) * CH
            pltpu.make_async_copy(b_hbm.at[pl.ds(0, CH)], rows_v.at[b],
                                  semb).wait()
            pltpu.sync_copy(rows_v.at[b], o_hbm.at[pl.ds(base, CH)])
            nxt = 2 * g + b + 2

            @pl.when(nxt < NCH_L)
            def _():
                nbase = (w * NCH_L + nxt) * CH
                pltpu.sync_copy(s_hbm.at[pl.ds(nbase, CH)], sidx_v.at[b])
                pltpu.async_copy(a_hbm.at[sidx_v.at[b]], rows_v.at[b], sema)
                pltpu.sync_copy(d_hbm.at[pl.ds(nbase, CH)], didx_v.at[b])
        return 0

    lax.fori_loop(0, NCH_L // 2, body, 0)


_BN = 1024


def _tc_call(body, grid, in_specs, out_specs, out_shape):
    return pl.pallas_call(body, grid=grid, in_specs=in_specs,
                          out_specs=out_specs, out_shape=out_shape)


def _dinv_of(deg_ref):
    deg = deg_ref[0, :] + deg_ref[1, :] + 1.0
    return lax.rsqrt(deg)


def _tc_h1p(xp, w1, degp):
    def body(x_ref, w_ref, deg_ref, o_ref):
        dinv = _dinv_of(deg_ref)
        h = jnp.dot(x_ref[...], w_ref[...], preferred_element_type=jnp.float32)
        o_ref[...] = h * dinv[:, None]

    return _tc_call(
        body, (NP_ // _BN,),
        [pl.BlockSpec((_BN, D), lambda i: (i, 0)),
         pl.BlockSpec((D, H), lambda i: (0, 0)),
         pl.BlockSpec((2, _BN), lambda i: (0, i))],
        pl.BlockSpec((_BN, H), lambda i: (i, 0)),
        jax.ShapeDtypeStruct((NP_, H), jnp.float32),
    )(xp, w1, degp)


def _tc_mid(parts, degp, b1r, w2):
    def body(p_ref, deg_ref, b_ref, w_ref, o_ref):
        dinv = _dinv_of(deg_ref)
        z = (p_ref[0] + p_ref[1]) * dinv[:, None] + b_ref[...]
        z = jnp.maximum(z, 0.0)
        h = jnp.dot(z, w_ref[...], preferred_element_type=jnp.float32)
        o_ref[...] = h * dinv[:, None]

    return _tc_call(
        body, (NP_ // _BN,),
        [pl.BlockSpec((2, _BN, H), lambda i: (0, i, 0)),
         pl.BlockSpec((2, _BN), lambda i: (0, i)),
         pl.BlockSpec((1, H), lambda i: (0, 0)),
         pl.BlockSpec((H, H), lambda i: (0, 0))],
        pl.BlockSpec((_BN, H), lambda i: (i, 0)),
        jax.ShapeDtypeStruct((NP_, H), jnp.float32),
    )(parts, degp, b1r, w2)


def _tc_ab(parts, degp, b2r, bc1r, wc1t, wc1b):
    def body(p_ref, deg_ref, b_ref, bc_ref, wt_ref, wb_ref, a_ref, bo_ref):
        dinv = _dinv_of(deg_ref)
        z = (p_ref[0] + p_ref[1]) * dinv[:, None] + b_ref[...]
        a_ref[...] = (jnp.dot(z, wt_ref[...], preferred_element_type=jnp.float32)
                      + bc_ref[...])
        bo_ref[...] = jnp.dot(z, wb_ref[...], preferred_element_type=jnp.float32)

    return _tc_call(
        body, (NP_ // _BN,),
        [pl.BlockSpec((2, _BN, H), lambda i: (0, i, 0)),
         pl.BlockSpec((2, _BN), lambda i: (0, i)),
         pl.BlockSpec((1, H), lambda i: (0, 0)),
         pl.BlockSpec((1, H), lambda i: (0, 0)),
         pl.BlockSpec((H, H), lambda i: (0, 0)),
         pl.BlockSpec((H, H), lambda i: (0, 0))],
        [pl.BlockSpec((_BN, H), lambda i: (i, 0)),
         pl.BlockSpec((_BN, H), lambda i: (i, 0))],
        [jax.ShapeDtypeStruct((NP_, H), jnp.float32),
         jax.ShapeDtypeStruct((NP_, H), jnp.float32)],
    )(parts, degp, b2r, bc1r, wc1t, wc1b)


def _tc_out(hsum, wc2t):
    def body(h_ref, w_ref, o_ref):
        h = jnp.maximum(h_ref[...], 0.0)
        o_ref[...] = lax.dot_general(
            w_ref[...], h, (((1,), (1,)), ((), ())),
            preferred_element_type=jnp.float32)

    return _tc_call(
        body, (ELP // _BN,),
        [pl.BlockSpec((_BN, H), lambda i: (i, 0)),
         pl.BlockSpec((8, H), lambda i: (0, 0))],
        pl.BlockSpec((8, _BN), lambda i: (0, i)),
        jax.ShapeDtypeStruct((8, ELP), jnp.float32),
    )(hsum, wc2t)


def kernel(x, edge_index, edge_label_index, W1, b1, W2, b2, Wc1, bc1, Wc2,
           bc2):
    f32 = jnp.float32
    xp = jnp.pad(x, ((0, NP_ - N), (0, 0)))
    pad_e = (jnp.arange(EP - E, dtype=jnp.int32) % (NP_ - N)) + N
    srcp = jnp.concatenate([edge_index[0], pad_e])
    dstp = jnp.concatenate([edge_index[1], pad_e])
    pad_l = (jnp.arange(ELP - EL, dtype=jnp.int32) % (NP_ - N)) + N
    sp = jnp.concatenate([edge_label_index[0], pad_l])
    dp = jnp.concatenate([edge_label_index[1], pad_l])

    b1r = b1.reshape(1, H)
    b2r = b2.reshape(1, H)
    bc1r = bc1.reshape(1, H)
    wc1t = Wc1[:H]
    wc1b = Wc1[H:]
    wc2t = jnp.pad(Wc2.T, ((0, 6), (0, 0)))

    degp = _sc_degree(dstp).reshape(2, NP_)

    h1p = _tc_h1p(xp, W1, degp)
    parts1 = _sc_spmm(srcp, dstp, h1p).reshape(2, NP_, D)
    h2p = _tc_mid(parts1, degp, b1r, W2)
    parts2 = _sc_spmm(srcp, dstp, h2p).reshape(2, NP_, D)
    a_nodes, b_nodes = _tc_ab(parts2, degp, b2r, bc1r, wc1t, wc1b)
    hsum = _sc_decode(sp, dp, a_nodes, b_nodes)
    outp = _tc_out(hsum, wc2t)
    return (outp[:2, :EL].T + bc2[None, :]).astype(f32)
